# Initial kernel scaffold; baseline (speedup 1.0000x reference)
import functools

import jax
import jax.numpy as jnp
from jax.experimental import pallas as pl
from jax.experimental.pallas import tpu as pltpu
from jax.experimental.pallas import tpu_sc as plsc

N = 10000
E = 320000
DIN = 128
DH = 128
NCLASS = 64

NC = 2
NS = 16
L = 16

CHUNK = 80
EPT = E // (NC * NS)
NCHUNK = EPT // CHUNK
ROWS_PER_SUB = N // NS
WCHUNK = 125


def _segsum_sc(h, edge_index, edge_weight, D):
  mesh = plsc.VectorSubcoreMesh(core_axis_name="c", subcore_axis_name="s")

  @functools.partial(
      pl.kernel,
      out_type=jax.ShapeDtypeStruct((NC, N, D), jnp.float32),
      mesh=mesh,
      scratch_types=[
          pltpu.VMEM((CHUNK,), jnp.int32),
          pltpu.VMEM((CHUNK,), jnp.int32),
          pltpu.VMEM((CHUNK,), jnp.float32),
          pltpu.VMEM((CHUNK, D), jnp.float32),
          pltpu.VMEM((WCHUNK, D), jnp.float32),
          pltpu.VMEM_SHARED((N, D), jnp.float32),
      ],
  )
  def k(h_hbm, ei_hbm, w_hbm, out_hbm, src_v, dst_v, w_v, rows_v, zero_v,
        acc_sh):
    c = jax.lax.axis_index("c")
    s = jax.lax.axis_index("s")

    zv = jnp.zeros((L,), jnp.float32)

    @pl.loop(0, WCHUNK)
    def _(r):
      for j in range(D // L):
        zero_v[r, pl.ds(j * L, L)] = zv

    @pl.loop(0, ROWS_PER_SUB // WCHUNK)
    def _(t):
      base = s * ROWS_PER_SUB + t * WCHUNK
      pltpu.sync_copy(zero_v, acc_sh.at[pl.ds(base, WCHUNK)])

    plsc.subcore_barrier()

    wid = c * NS + s
    tile_base = wid * EPT

    @pl.loop(0, NCHUNK)
    def _(t):
      base = tile_base + t * CHUNK
      pltpu.sync_copy(ei_hbm.at[0, pl.ds(base, CHUNK)], src_v)
      pltpu.sync_copy(ei_hbm.at[1, pl.ds(base, CHUNK)], dst_v)
      pltpu.sync_copy(w_hbm.at[pl.ds(base, CHUNK)], w_v)
      pltpu.sync_copy(h_hbm.at[src_v], rows_v)

      @pl.loop(0, CHUNK)
      def _(r):
        wvec = plsc.load_gather(w_v, [jnp.full((L,), r, jnp.int32)])
        for j in range(D // L):
          sl = pl.ds(j * L, L)
          rows_v[r, sl] = rows_v[r, sl] * wvec

      pltpu.sync_copy(rows_v, acc_sh.at[dst_v], add=True)

    plsc.subcore_barrier()

    @pl.loop(0, ROWS_PER_SUB // WCHUNK)
    def _(t):
      base = s * ROWS_PER_SUB + t * WCHUNK
      pltpu.sync_copy(acc_sh.at[pl.ds(base, WCHUNK)],
                      out_hbm.at[c, pl.ds(base, WCHUNK)])

  return k(h, edge_index, edge_weight)


BM = 1000


def _mm_tc(x, W):
  M, K = x.shape
  Kw, Do = W.shape

  def kern(x_ref, w_ref, o_ref):
    o_ref[...] = jnp.dot(x_ref[...], w_ref[...],
                         preferred_element_type=jnp.float32)

  return pl.pallas_call(
      kern,
      grid=(M // BM,),
      in_specs=[
          pl.BlockSpec((BM, K), lambda i: (i, 0)),
          pl.BlockSpec((Kw, Do), lambda i: (0, 0)),
      ],
      out_specs=pl.BlockSpec((BM, Do), lambda i: (i, 0)),
      out_shape=jax.ShapeDtypeStruct((M, Do), jnp.float32),
  )(x, W)


def _fuse_tc(p, b, g, bb, W):
  _, M, D = p.shape
  Dw, Do = W.shape

  def kern(p_ref, b_ref, g_ref, bb_ref, w_ref, o_ref):
    x = p_ref[0] + p_ref[1] + b_ref[...]
    mu = jnp.mean(x, axis=-1, keepdims=True)
    var = jnp.mean(jnp.square(x - mu), axis=-1, keepdims=True)
    x = (x - mu) * jax.lax.rsqrt(var + 1e-5) * g_ref[...] + bb_ref[...]
    x = jnp.maximum(x, 0.0)
    o_ref[...] = jnp.dot(x, w_ref[...], preferred_element_type=jnp.float32)

  return pl.pallas_call(
      kern,
      grid=(M // BM,),
      in_specs=[
          pl.BlockSpec((2, BM, D), lambda i: (0, i, 0)),
          pl.BlockSpec((1, D), lambda i: (0, 0)),
          pl.BlockSpec((1, D), lambda i: (0, 0)),
          pl.BlockSpec((1, D), lambda i: (0, 0)),
          pl.BlockSpec((Dw, Do), lambda i: (0, 0)),
      ],
      out_specs=pl.BlockSpec((BM, Do), lambda i: (i, 0)),
      out_shape=jax.ShapeDtypeStruct((M, Do), jnp.float32),
  )(p, b, g, bb, W)


def _final_tc(p, b):
  _, M, D = p.shape

  def kern(p_ref, b_ref, o_ref):
    x = p_ref[0] + p_ref[1] + b_ref[...]
    m = jnp.max(x, axis=-1, keepdims=True)
    e = jnp.exp(x - m)
    lse = jnp.log(jnp.sum(e, axis=-1, keepdims=True)) + m
    o_ref[...] = x - lse

  return pl.pallas_call(
      kern,
      grid=(M // BM,),
      in_specs=[
          pl.BlockSpec((2, BM, D), lambda i: (0, i, 0)),
          pl.BlockSpec((1, D), lambda i: (0, 0)),
      ],
      out_specs=pl.BlockSpec((BM, D), lambda i: (i, 0)),
      out_shape=jax.ShapeDtypeStruct((M, D), jnp.float32),
  )(p, b)


def kernel(feats, edge_index, edge_weight, W1, b1, W2, b2, W3, b3, W4, b4,
           ln_g, ln_b):
  b1r = b1.reshape(1, DH)
  b2r = b2.reshape(1, DH)
  b3r = b3.reshape(1, DH)
  b4r = b4.reshape(1, NCLASS)
  gr = ln_g.reshape(1, DH)
  br = ln_b.reshape(1, DH)

  h = _mm_tc(feats, W1)
  p = _segsum_sc(h, edge_index, edge_weight, DH)
  h = _fuse_tc(p, b1r, gr, br, W2)
  p = _segsum_sc(h, edge_index, edge_weight, DH)
  h = _fuse_tc(p, b2r, gr, br, W3)
  p = _segsum_sc(h, edge_index, edge_weight, DH)
  h = _fuse_tc(p, b3r, gr, br, W4)
  p = _segsum_sc(h, edge_index, edge_weight, NCLASS)
  return _final_tc(p, b4r)


# speedup vs baseline: 3.4588x; 3.4588x over previous
import dataclasses
import functools

import jax
import jax.numpy as jnp
from jax.experimental import pallas as pl
from jax.experimental.pallas import tpu as pltpu
from jax.experimental.pallas import tpu_sc as plsc

N = 10000
E = 320000
DIN = 128
DH = 128
NCLASS = 64

NC = 2
NS = 16
L = 16

CHUNK = 80
EPT = E // (NC * NS)
NCHUNK = EPT // CHUNK
WCHUNK = 80
NWCHUNK = N // WCHUNK


def _segsum_sc(h, src_idx, dst_idx, edge_weight, D):
  mesh = plsc.VectorSubcoreMesh(core_axis_name="c", subcore_axis_name="s")
  cp = pltpu.CompilerParams()
  if "needs_layout_passes" in pltpu.CompilerParams.__dataclass_fields__:
    cp = dataclasses.replace(cp, needs_layout_passes=False)

  @functools.partial(
      pl.kernel,
      out_type=jax.ShapeDtypeStruct((NC, N, D), jnp.float32),
      mesh=mesh,
      scratch_types=[
          pltpu.VMEM((CHUNK,), jnp.int32),
          pltpu.VMEM((CHUNK,), jnp.int32),
          pltpu.VMEM((CHUNK,), jnp.float32),
          pltpu.VMEM((CHUNK, D), jnp.float32),
          pltpu.VMEM((WCHUNK, D), jnp.float32),
          pltpu.VMEM_SHARED((N, D), jnp.float32),
      ],
      compiler_params=cp,
  )
  def k(h_hbm, si_hbm, di_hbm, w_hbm, out_hbm, src_v, dst_v, w_v, rows_v,
        zero_v, acc_sh):
    c = jax.lax.axis_index("c")
    s = jax.lax.axis_index("s")

    zv = jnp.zeros((L,), jnp.float32)

    @pl.loop(0, WCHUNK)
    def _(r):
      for j in range(D // L):
        zero_v[r, pl.ds(j * L, L)] = zv

    @pl.loop(0, NWCHUNK)
    def _(j):
      @pl.when(j % NS == s)
      def _():
        pltpu.sync_copy(zero_v, acc_sh.at[pl.ds(j * WCHUNK, WCHUNK)])

    plsc.subcore_barrier()

    wid = c * NS + s
    tile_base = wid * EPT

    @pl.loop(0, NCHUNK)
    def _(t):
      base = tile_base + t * CHUNK
      pltpu.sync_copy(si_hbm.at[pl.ds(base, CHUNK)], src_v)
      pltpu.sync_copy(di_hbm.at[pl.ds(base, CHUNK)], dst_v)
      pltpu.sync_copy(w_hbm.at[pl.ds(base, CHUNK)], w_v)
      pltpu.sync_copy(h_hbm.at[src_v], rows_v)

      @pl.loop(0, CHUNK)
      def _(r):
        wvec = plsc.load_gather(w_v, [jnp.full((L,), r, jnp.int32)])
        for j in range(D // L):
          sl = pl.ds(j * L, L)
          rows_v[r, sl] = rows_v[r, sl] * wvec

      pltpu.sync_copy(rows_v, acc_sh.at[dst_v], add=True)

    plsc.subcore_barrier()

    @pl.loop(0, NWCHUNK)
    def _(j):
      @pl.when(j % NS == s)
      def _():
        base = j * WCHUNK
        pltpu.sync_copy(acc_sh.at[pl.ds(base, WCHUNK)],
                        out_hbm.at[c, pl.ds(base, WCHUNK)])

  return k(h, src_idx, dst_idx, edge_weight)


BM = 1000


def _mm_tc(x, W):
  M, K = x.shape
  Kw, Do = W.shape

  def kern(x_ref, w_ref, o_ref):
    o_ref[...] = jnp.dot(x_ref[...], w_ref[...],
                         preferred_element_type=jnp.float32)

  return pl.pallas_call(
      kern,
      grid=(M // BM,),
      in_specs=[
          pl.BlockSpec((BM, K), lambda i: (i, 0)),
          pl.BlockSpec((Kw, Do), lambda i: (0, 0)),
      ],
      out_specs=pl.BlockSpec((BM, Do), lambda i: (i, 0)),
      out_shape=jax.ShapeDtypeStruct((M, Do), jnp.float32),
  )(x, W)


def _fuse_tc(p, b, g, bb, W):
  _, M, D = p.shape
  Dw, Do = W.shape

  def kern(p_ref, b_ref, g_ref, bb_ref, w_ref, o_ref):
    x = p_ref[0] + p_ref[1] + b_ref[...]
    mu = jnp.mean(x, axis=-1, keepdims=True)
    var = jnp.mean(jnp.square(x - mu), axis=-1, keepdims=True)
    x = (x - mu) * jax.lax.rsqrt(var + 1e-5) * g_ref[...] + bb_ref[...]
    x = jnp.maximum(x, 0.0)
    o_ref[...] = jnp.dot(x, w_ref[...], preferred_element_type=jnp.float32)

  return pl.pallas_call(
      kern,
      grid=(M // BM,),
      in_specs=[
          pl.BlockSpec((2, BM, D), lambda i: (0, i, 0)),
          pl.BlockSpec((1, D), lambda i: (0, 0)),
          pl.BlockSpec((1, D), lambda i: (0, 0)),
          pl.BlockSpec((1, D), lambda i: (0, 0)),
          pl.BlockSpec((Dw, Do), lambda i: (0, 0)),
      ],
      out_specs=pl.BlockSpec((BM, Do), lambda i: (i, 0)),
      out_shape=jax.ShapeDtypeStruct((M, Do), jnp.float32),
  )(p, b, g, bb, W)


def _final_tc(p, b):
  _, M, Dp = p.shape
  D = NCLASS

  def kern(p_ref, b_ref, o_ref):
    x = p_ref[0, :, :D] + p_ref[1, :, :D] + b_ref[...]
    m = jnp.max(x, axis=-1, keepdims=True)
    e = jnp.exp(x - m)
    lse = jnp.log(jnp.sum(e, axis=-1, keepdims=True)) + m
    o_ref[...] = x - lse

  return pl.pallas_call(
      kern,
      grid=(M // BM,),
      in_specs=[
          pl.BlockSpec((2, BM, Dp), lambda i: (0, i, 0)),
          pl.BlockSpec((1, D), lambda i: (0, 0)),
      ],
      out_specs=pl.BlockSpec((BM, D), lambda i: (i, 0)),
      out_shape=jax.ShapeDtypeStruct((M, D), jnp.float32),
  )(p, b)


def kernel(feats, edge_index, edge_weight, W1, b1, W2, b2, W3, b3, W4, b4,
           ln_g, ln_b):
  b1r = b1.reshape(1, DH)
  b2r = b2.reshape(1, DH)
  b3r = b3.reshape(1, DH)
  b4r = b4.reshape(1, NCLASS)
  gr = ln_g.reshape(1, DH)
  br = ln_b.reshape(1, DH)

  src_idx = edge_index[0]
  dst_idx = edge_index[1]

  h = _mm_tc(feats, W1)
  p = _segsum_sc(h, src_idx, dst_idx, edge_weight, DH)
  h = _fuse_tc(p, b1r, gr, br, W2)
  p = _segsum_sc(h, src_idx, dst_idx, edge_weight, DH)
  h = _fuse_tc(p, b2r, gr, br, W3)
  p = _segsum_sc(h, src_idx, dst_idx, edge_weight, DH)
  W4p = jnp.pad(W4, ((0, 0), (0, DH - NCLASS)))
  h = _fuse_tc(p, b3r, gr, br, W4p)
  p = _segsum_sc(h, src_idx, dst_idx, edge_weight, DH)
  return _final_tc(p, b4r)


# speedup vs baseline: 7.0795x; 2.0468x over previous
import dataclasses
import functools

import jax
import jax.numpy as jnp
from jax.experimental import pallas as pl
from jax.experimental.pallas import tpu as pltpu
from jax.experimental.pallas import tpu_sc as plsc

N = 10000
E = 320000
DIN = 128
DH = 128
NCLASS = 64

NC = 2
NS = 16
L = 16

NT = NC * NS
CHUNK = 125
EPT = E // NT
NCHUNK = EPT // CHUNK
WCHUNK = 80
NWCHUNK = N // WCHUNK


def _segsum_sc(h, esd, D):
  mesh = plsc.VectorSubcoreMesh(core_axis_name="c", subcore_axis_name="s")
  cp = pltpu.CompilerParams()
  if "needs_layout_passes" in pltpu.CompilerParams.__dataclass_fields__:
    cp = dataclasses.replace(cp, needs_layout_passes=False)

  @functools.partial(
      pl.kernel,
      out_type=jax.ShapeDtypeStruct((NC, N, D), jnp.float32),
      mesh=mesh,
      scratch_types=[
          pltpu.VMEM((3, CHUNK), jnp.int32),
          pltpu.VMEM((3, CHUNK), jnp.int32),
          pltpu.VMEM((CHUNK, D), jnp.float32),
          pltpu.VMEM((CHUNK, D), jnp.float32),
          pltpu.VMEM_SHARED((N, D), jnp.float32),
          pltpu.SemaphoreType.DMA,
          pltpu.SemaphoreType.DMA,
      ],
      compiler_params=cp,
  )
  def k(h_hbm, esd_hbm, out_hbm, ibuf0_v, ibuf1_v, rows0_v, rows1_v, acc_sh,
        gsem0, gsem1):
    c = jax.lax.axis_index("c")
    s = jax.lax.axis_index("s")
    wid = c * NS + s

    zv = jnp.zeros((L,), jnp.float32)

    @pl.loop(0, WCHUNK)
    def _(r):
      for j in range(D // L):
        rows0_v[r, pl.ds(j * L, L)] = zv

    @pl.loop(0, NWCHUNK)
    def _(j):
      @pl.when(j % NS == s)
      def _():
        pltpu.sync_copy(rows0_v.at[pl.ds(0, WCHUNK)],
                        acc_sh.at[pl.ds(j * WCHUNK, WCHUNK)])

    plsc.subcore_barrier()

    rows = (rows0_v, rows1_v)
    ibufs = (ibuf0_v, ibuf1_v)
    gsem = (gsem0, gsem1)

    def idx_copy(t, b):
      pltpu.sync_copy(esd_hbm.at[wid, t], ibufs[b])

    def gather_start(t, b):
      del t
      pltpu.async_copy(h_hbm.at[ibufs[b].at[0]], rows[b], gsem[b])

    def gather_wait(t, b):
      del t
      pltpu.make_async_copy(h_hbm.at[ibufs[b].at[0]], rows[b], gsem[b]).wait()

    def scale_scatter(t, b):
      del t
      rv = rows[b]
      ib = ibufs[b]
      two = jnp.full((L,), 2, jnp.int32)

      @pl.loop(0, CHUNK)
      def _(r):
        wvec = plsc.bitcast(
            plsc.load_gather(ib, [two, jnp.full((L,), r, jnp.int32)]),
            jnp.float32)
        for j in range(D // L):
          sl = pl.ds(j * L, L)
          rv[r, sl] = rv[r, sl] * wvec

      pltpu.sync_copy(rv, acc_sh.at[ib.at[1]], add=True)

    idx_copy(0, 0)
    gather_start(0, 0)

    @pl.loop(0, NCHUNK // 2)
    def _(u):
      t0 = u * 2
      idx_copy(t0 + 1, 1)
      gather_start(t0 + 1, 1)
      gather_wait(t0, 0)
      scale_scatter(t0, 0)

      @pl.when(u < NCHUNK // 2 - 1)
      def _():
        idx_copy(t0 + 2, 0)
        gather_start(t0 + 2, 0)

      gather_wait(t0 + 1, 1)
      scale_scatter(t0 + 1, 1)

    plsc.subcore_barrier()

    @pl.loop(0, NWCHUNK)
    def _(j):
      @pl.when(j % NS == s)
      def _():
        base = j * WCHUNK
        pltpu.sync_copy(acc_sh.at[pl.ds(base, WCHUNK)],
                        out_hbm.at[c, pl.ds(base, WCHUNK)])

  return k(h, esd)


BM = 1000


def _mm_tc(x, W):
  M, K = x.shape
  Kw, Do = W.shape

  def kern(x_ref, w_ref, o_ref):
    o_ref[...] = jnp.dot(x_ref[...], w_ref[...],
                         preferred_element_type=jnp.float32)

  return pl.pallas_call(
      kern,
      grid=(M // BM,),
      in_specs=[
          pl.BlockSpec((BM, K), lambda i: (i, 0)),
          pl.BlockSpec((Kw, Do), lambda i: (0, 0)),
      ],
      out_specs=pl.BlockSpec((BM, Do), lambda i: (i, 0)),
      out_shape=jax.ShapeDtypeStruct((M, Do), jnp.float32),
  )(x, W)


def _fuse_tc(p, b, g, bb, W):
  _, M, D = p.shape
  Dw, Do = W.shape

  def kern(p_ref, b_ref, g_ref, bb_ref, w_ref, o_ref):
    x = p_ref[0] + p_ref[1] + b_ref[...]
    mu = jnp.mean(x, axis=-1, keepdims=True)
    var = jnp.mean(jnp.square(x - mu), axis=-1, keepdims=True)
    x = (x - mu) * jax.lax.rsqrt(var + 1e-5) * g_ref[...] + bb_ref[...]
    x = jnp.maximum(x, 0.0)
    o_ref[...] = jnp.dot(x, w_ref[...], preferred_element_type=jnp.float32)

  return pl.pallas_call(
      kern,
      grid=(M // BM,),
      in_specs=[
          pl.BlockSpec((2, BM, D), lambda i: (0, i, 0)),
          pl.BlockSpec((1, D), lambda i: (0, 0)),
          pl.BlockSpec((1, D), lambda i: (0, 0)),
          pl.BlockSpec((1, D), lambda i: (0, 0)),
          pl.BlockSpec((Dw, Do), lambda i: (0, 0)),
      ],
      out_specs=pl.BlockSpec((BM, Do), lambda i: (i, 0)),
      out_shape=jax.ShapeDtypeStruct((M, Do), jnp.float32),
  )(p, b, g, bb, W)


def _final_tc(p, b):
  _, M, Dp = p.shape
  D = NCLASS

  def kern(p_ref, b_ref, o_ref):
    x = p_ref[0, :, :D] + p_ref[1, :, :D] + b_ref[...]
    m = jnp.max(x, axis=-1, keepdims=True)
    e = jnp.exp(x - m)
    lse = jnp.log(jnp.sum(e, axis=-1, keepdims=True)) + m
    o_ref[...] = x - lse

  return pl.pallas_call(
      kern,
      grid=(M // BM,),
      in_specs=[
          pl.BlockSpec((2, BM, Dp), lambda i: (0, i, 0)),
          pl.BlockSpec((1, D), lambda i: (0, 0)),
      ],
      out_specs=pl.BlockSpec((BM, D), lambda i: (i, 0)),
      out_shape=jax.ShapeDtypeStruct((M, D), jnp.float32),
  )(p, b)


def kernel(feats, edge_index, edge_weight, W1, b1, W2, b2, W3, b3, W4, b4,
           ln_g, ln_b):
  b1r = b1.reshape(1, DH)
  b2r = b2.reshape(1, DH)
  b3r = b3.reshape(1, DH)
  b4r = b4.reshape(1, NCLASS)
  gr = ln_g.reshape(1, DH)
  br = ln_b.reshape(1, DH)

  src_idx = edge_index[0].reshape(NT, NCHUNK, CHUNK)
  dst_idx = edge_index[1].reshape(NT, NCHUNK, CHUNK)
  wbits = jax.lax.bitcast_convert_type(
      edge_weight, jnp.int32).reshape(NT, NCHUNK, CHUNK)
  esd = jnp.stack([src_idx, dst_idx, wbits], axis=2)

  h = _mm_tc(feats, W1)
  p = _segsum_sc(h, esd, DH)
  h = _fuse_tc(p, b1r, gr, br, W2)
  p = _segsum_sc(h, esd, DH)
  h = _fuse_tc(p, b2r, gr, br, W3)
  p = _segsum_sc(h, esd, DH)
  W4p = jnp.pad(W4, ((0, 0), (0, DH - NCLASS)))
  h = _fuse_tc(p, b3r, gr, br, W4p)
  p = _segsum_sc(h, esd, DH)
  return _final_tc(p, b4r)


# speedup vs baseline: 8.9354x; 1.2622x over previous
import dataclasses
import functools

import jax
import jax.numpy as jnp
from jax.experimental import pallas as pl
from jax.experimental.pallas import tpu as pltpu
from jax.experimental.pallas import tpu_sc as plsc

N = 10000
E = 320000
DIN = 128
DH = 128
NCLASS = 64

NC = 2
NS = 16
L = 16

NT = NC * NS
CHUNK = 125
EPT = E // NT
NCHUNK = EPT // CHUNK
WCHUNK = 80
NWCHUNK = N // WCHUNK


def _segsum_sc(h, esd, D):
  mesh = plsc.VectorSubcoreMesh(core_axis_name="c", subcore_axis_name="s")
  cp = pltpu.CompilerParams()
  if "needs_layout_passes" in pltpu.CompilerParams.__dataclass_fields__:
    cp = dataclasses.replace(cp, needs_layout_passes=False)

  @functools.partial(
      pl.kernel,
      out_type=jax.ShapeDtypeStruct((NC, N, D), jnp.float32),
      mesh=mesh,
      scratch_types=[
          pltpu.VMEM((3, CHUNK), jnp.int32),
          pltpu.VMEM((3, CHUNK), jnp.int32),
          pltpu.VMEM((CHUNK, D), jnp.float32),
          pltpu.VMEM((CHUNK, D), jnp.float32),
          pltpu.VMEM_SHARED((N, D), jnp.float32),
          pltpu.SemaphoreType.DMA,
          pltpu.SemaphoreType.DMA,
      ],
      compiler_params=cp,
  )
  def k(h_hbm, esd_hbm, out_hbm, ibuf0_v, ibuf1_v, rows0_v, rows1_v, acc_sh,
        gsem0, gsem1):
    c = jax.lax.axis_index("c")
    s = jax.lax.axis_index("s")
    wid = c * NS + s

    zv = jnp.zeros((L,), jnp.float32)

    @pl.loop(0, WCHUNK)
    def _(r):
      for j in range(D // L):
        rows0_v[r, pl.ds(j * L, L)] = zv

    @pl.loop(0, NWCHUNK)
    def _(j):
      @pl.when(j % NS == s)
      def _():
        pltpu.sync_copy(rows0_v.at[pl.ds(0, WCHUNK)],
                        acc_sh.at[pl.ds(j * WCHUNK, WCHUNK)])

    plsc.subcore_barrier()

    rows = (rows0_v, rows1_v)
    ibufs = (ibuf0_v, ibuf1_v)
    gsem = (gsem0, gsem1)

    def idx_copy(t, b):
      pltpu.sync_copy(esd_hbm.at[wid, t], ibufs[b])

    def gather_start(t, b):
      del t
      pltpu.async_copy(h_hbm.at[ibufs[b].at[0]], rows[b], gsem[b])

    def gather_wait(t, b):
      del t
      pltpu.make_async_copy(h_hbm.at[ibufs[b].at[0]], rows[b], gsem[b]).wait()

    def scale_scatter(t, b):
      del t
      rv = rows[b]
      ib = ibufs[b]

      @plsc.parallel_loop(0, CHUNK, unroll=5)
      def _(r):
        wvec = plsc.bitcast(
            plsc.load_gather(ib, [jnp.full((L,), 2, jnp.int32),
                                  jnp.full((L,), r, jnp.int32)]),
            jnp.float32)
        for j in range(D // L):
          sl = pl.ds(j * L, L)
          rv[r, sl] = rv[r, sl] * wvec

      pltpu.sync_copy(rv, acc_sh.at[ib.at[1]], add=True)

    idx_copy(0, 0)
    gather_start(0, 0)

    @pl.loop(0, NCHUNK // 2)
    def _(u):
      t0 = u * 2
      idx_copy(t0 + 1, 1)
      gather_start(t0 + 1, 1)
      gather_wait(t0, 0)
      scale_scatter(t0, 0)

      @pl.when(u < NCHUNK // 2 - 1)
      def _():
        idx_copy(t0 + 2, 0)
        gather_start(t0 + 2, 0)

      gather_wait(t0 + 1, 1)
      scale_scatter(t0 + 1, 1)

    plsc.subcore_barrier()

    @pl.loop(0, NWCHUNK)
    def _(j):
      @pl.when(j % NS == s)
      def _():
        base = j * WCHUNK
        pltpu.sync_copy(acc_sh.at[pl.ds(base, WCHUNK)],
                        out_hbm.at[c, pl.ds(base, WCHUNK)])

  return k(h, esd)


BM = 1000


def _mm_tc(x, W):
  M, K = x.shape
  Kw, Do = W.shape

  def kern(x_ref, w_ref, o_ref):
    o_ref[...] = jnp.dot(x_ref[...], w_ref[...],
                         preferred_element_type=jnp.float32)

  return pl.pallas_call(
      kern,
      grid=(M // BM,),
      in_specs=[
          pl.BlockSpec((BM, K), lambda i: (i, 0)),
          pl.BlockSpec((Kw, Do), lambda i: (0, 0)),
      ],
      out_specs=pl.BlockSpec((BM, Do), lambda i: (i, 0)),
      out_shape=jax.ShapeDtypeStruct((M, Do), jnp.float32),
  )(x, W)


def _fuse_tc(p, b, g, bb, W):
  _, M, D = p.shape
  Dw, Do = W.shape

  def kern(p_ref, b_ref, g_ref, bb_ref, w_ref, o_ref):
    x = p_ref[0] + p_ref[1] + b_ref[...]
    mu = jnp.mean(x, axis=-1, keepdims=True)
    var = jnp.mean(jnp.square(x - mu), axis=-1, keepdims=True)
    x = (x - mu) * jax.lax.rsqrt(var + 1e-5) * g_ref[...] + bb_ref[...]
    x = jnp.maximum(x, 0.0)
    o_ref[...] = jnp.dot(x, w_ref[...], preferred_element_type=jnp.float32)

  return pl.pallas_call(
      kern,
      grid=(M // BM,),
      in_specs=[
          pl.BlockSpec((2, BM, D), lambda i: (0, i, 0)),
          pl.BlockSpec((1, D), lambda i: (0, 0)),
          pl.BlockSpec((1, D), lambda i: (0, 0)),
          pl.BlockSpec((1, D), lambda i: (0, 0)),
          pl.BlockSpec((Dw, Do), lambda i: (0, 0)),
      ],
      out_specs=pl.BlockSpec((BM, Do), lambda i: (i, 0)),
      out_shape=jax.ShapeDtypeStruct((M, Do), jnp.float32),
  )(p, b, g, bb, W)


def _final_tc(p, b):
  _, M, Dp = p.shape
  D = NCLASS

  def kern(p_ref, b_ref, o_ref):
    x = p_ref[0, :, :D] + p_ref[1, :, :D] + b_ref[...]
    m = jnp.max(x, axis=-1, keepdims=True)
    e = jnp.exp(x - m)
    lse = jnp.log(jnp.sum(e, axis=-1, keepdims=True)) + m
    o_ref[...] = x - lse

  return pl.pallas_call(
      kern,
      grid=(M // BM,),
      in_specs=[
          pl.BlockSpec((2, BM, Dp), lambda i: (0, i, 0)),
          pl.BlockSpec((1, D), lambda i: (0, 0)),
      ],
      out_specs=pl.BlockSpec((BM, D), lambda i: (i, 0)),
      out_shape=jax.ShapeDtypeStruct((M, D), jnp.float32),
  )(p, b)


def kernel(feats, edge_index, edge_weight, W1, b1, W2, b2, W3, b3, W4, b4,
           ln_g, ln_b):
  b1r = b1.reshape(1, DH)
  b2r = b2.reshape(1, DH)
  b3r = b3.reshape(1, DH)
  b4r = b4.reshape(1, NCLASS)
  gr = ln_g.reshape(1, DH)
  br = ln_b.reshape(1, DH)

  src_idx = edge_index[0].reshape(NT, NCHUNK, CHUNK)
  dst_idx = edge_index[1].reshape(NT, NCHUNK, CHUNK)
  wbits = jax.lax.bitcast_convert_type(
      edge_weight, jnp.int32).reshape(NT, NCHUNK, CHUNK)
  esd = jnp.stack([src_idx, dst_idx, wbits], axis=2)

  h = _mm_tc(feats, W1)
  p = _segsum_sc(h, esd, DH)
  h = _fuse_tc(p, b1r, gr, br, W2)
  p = _segsum_sc(h, esd, DH)
  h = _fuse_tc(p, b2r, gr, br, W3)
  p = _segsum_sc(h, esd, DH)
  W4p = jnp.pad(W4, ((0, 0), (0, DH - NCLASS)))
  h = _fuse_tc(p, b3r, gr, br, W4p)
  p = _segsum_sc(h, esd, DH)
  return _final_tc(p, b4r)


# speedup vs baseline: 9.0903x; 1.0173x over previous
import dataclasses
import functools

import jax
import jax.numpy as jnp
from jax.experimental import pallas as pl
from jax.experimental.pallas import tpu as pltpu
from jax.experimental.pallas import tpu_sc as plsc

N = 10000
E = 320000
DIN = 128
DH = 128
NCLASS = 64

NC = 2
NS = 16
L = 16

NT = NC * NS
CHUNK = 125
EPT = E // NT
NCHUNK = EPT // CHUNK
WCHUNK = 80
NWCHUNK = N // WCHUNK


def _segsum_sc(h, esd, D):
  mesh = plsc.VectorSubcoreMesh(core_axis_name="c", subcore_axis_name="s")
  cp = pltpu.CompilerParams()
  if "needs_layout_passes" in pltpu.CompilerParams.__dataclass_fields__:
    cp = dataclasses.replace(cp, needs_layout_passes=False)

  @functools.partial(
      pl.kernel,
      out_type=jax.ShapeDtypeStruct((NC, N, D), jnp.float32),
      mesh=mesh,
      scratch_types=[
          pltpu.VMEM((3, CHUNK), jnp.int32),
          pltpu.VMEM((3, CHUNK), jnp.int32),
          pltpu.VMEM((CHUNK, D), jnp.float32),
          pltpu.VMEM((CHUNK, D), jnp.float32),
          pltpu.VMEM_SHARED((N, D), jnp.float32),
          pltpu.SemaphoreType.DMA,
          pltpu.SemaphoreType.DMA,
          pltpu.SemaphoreType.DMA,
          pltpu.SemaphoreType.DMA,
      ],
      compiler_params=cp,
  )
  def k(h_hbm, esd_hbm, out_hbm, ibuf0_v, ibuf1_v, rows0_v, rows1_v, acc_sh,
        gsem0, gsem1, ssem0, ssem1):
    c = jax.lax.axis_index("c")
    s = jax.lax.axis_index("s")
    wid = c * NS + s

    zv = jnp.zeros((L,), jnp.float32)

    @pl.loop(0, WCHUNK)
    def _(r):
      for j in range(D // L):
        rows0_v[r, pl.ds(j * L, L)] = zv

    @pl.loop(0, NWCHUNK)
    def _(j):
      @pl.when(j % NS == s)
      def _():
        pltpu.sync_copy(rows0_v.at[pl.ds(0, WCHUNK)],
                        acc_sh.at[pl.ds(j * WCHUNK, WCHUNK)])

    plsc.subcore_barrier()

    rows = (rows0_v, rows1_v)
    ibufs = (ibuf0_v, ibuf1_v)
    gsem = (gsem0, gsem1)
    ssem = (ssem0, ssem1)

    def idx_copy(t, b):
      pltpu.sync_copy(esd_hbm.at[wid, t], ibufs[b])

    def gather_start(b):
      pltpu.async_copy(h_hbm.at[ibufs[b].at[0]], rows[b], gsem[b])

    def gather_wait(b):
      pltpu.make_async_copy(h_hbm.at[ibufs[b].at[0]], rows[b], gsem[b]).wait()

    def scale(b):
      rv = rows[b]
      ib = ibufs[b]

      @plsc.parallel_loop(0, CHUNK, unroll=5)
      def _(r):
        wvec = plsc.bitcast(
            plsc.load_gather(ib, [jnp.full((L,), 2, jnp.int32),
                                  jnp.full((L,), r, jnp.int32)]),
            jnp.float32)
        for j in range(D // L):
          sl = pl.ds(j * L, L)
          rv[r, sl] = rv[r, sl] * wvec

    def scatter_start(b):
      pltpu.async_copy(rows[b], acc_sh.at[ibufs[b].at[1]], ssem[b], add=True)

    def scatter_wait(b):
      pltpu.make_async_copy(rows[b], acc_sh.at[ibufs[b].at[1]],
                            ssem[b]).wait()

    idx_copy(0, 0)
    gather_start(0)
    idx_copy(1, 1)
    gather_start(1)

    @pl.loop(0, NCHUNK // 2)
    def _(u):
      t0 = u * 2
      gather_wait(0)
      scale(0)
      scatter_start(0)
      gather_wait(1)
      scale(1)
      scatter_start(1)

      @pl.when(u < NCHUNK // 2 - 1)
      def _():
        scatter_wait(0)
        idx_copy(t0 + 2, 0)
        gather_start(0)
        scatter_wait(1)
        idx_copy(t0 + 3, 1)
        gather_start(1)

    scatter_wait(0)
    scatter_wait(1)
    plsc.subcore_barrier()

    @pl.loop(0, NWCHUNK)
    def _(j):
      @pl.when(j % NS == s)
      def _():
        base = j * WCHUNK
        pltpu.sync_copy(acc_sh.at[pl.ds(base, WCHUNK)],
                        out_hbm.at[c, pl.ds(base, WCHUNK)])

  return k(h, esd)


BM = 1000


def _mm_tc(x, W):
  M, K = x.shape
  Kw, Do = W.shape

  def kern(x_ref, w_ref, o_ref):
    o_ref[...] = jnp.dot(x_ref[...], w_ref[...],
                         preferred_element_type=jnp.float32)

  return pl.pallas_call(
      kern,
      grid=(M // BM,),
      in_specs=[
          pl.BlockSpec((BM, K), lambda i: (i, 0)),
          pl.BlockSpec((Kw, Do), lambda i: (0, 0)),
      ],
      out_specs=pl.BlockSpec((BM, Do), lambda i: (i, 0)),
      out_shape=jax.ShapeDtypeStruct((M, Do), jnp.float32),
  )(x, W)


def _fuse_tc(p, b, g, bb, W):
  _, M, D = p.shape
  Dw, Do = W.shape

  def kern(p_ref, b_ref, g_ref, bb_ref, w_ref, o_ref):
    x = p_ref[0] + p_ref[1] + b_ref[...]
    mu = jnp.mean(x, axis=-1, keepdims=True)
    var = jnp.mean(jnp.square(x - mu), axis=-1, keepdims=True)
    x = (x - mu) * jax.lax.rsqrt(var + 1e-5) * g_ref[...] + bb_ref[...]
    x = jnp.maximum(x, 0.0)
    o_ref[...] = jnp.dot(x, w_ref[...], preferred_element_type=jnp.float32)

  return pl.pallas_call(
      kern,
      grid=(M // BM,),
      in_specs=[
          pl.BlockSpec((2, BM, D), lambda i: (0, i, 0)),
          pl.BlockSpec((1, D), lambda i: (0, 0)),
          pl.BlockSpec((1, D), lambda i: (0, 0)),
          pl.BlockSpec((1, D), lambda i: (0, 0)),
          pl.BlockSpec((Dw, Do), lambda i: (0, 0)),
      ],
      out_specs=pl.BlockSpec((BM, Do), lambda i: (i, 0)),
      out_shape=jax.ShapeDtypeStruct((M, Do), jnp.float32),
  )(p, b, g, bb, W)


def _final_tc(p, b):
  _, M, Dp = p.shape
  D = NCLASS

  def kern(p_ref, b_ref, o_ref):
    x = p_ref[0, :, :D] + p_ref[1, :, :D] + b_ref[...]
    m = jnp.max(x, axis=-1, keepdims=True)
    e = jnp.exp(x - m)
    lse = jnp.log(jnp.sum(e, axis=-1, keepdims=True)) + m
    o_ref[...] = x - lse

  return pl.pallas_call(
      kern,
      grid=(M // BM,),
      in_specs=[
          pl.BlockSpec((2, BM, Dp), lambda i: (0, i, 0)),
          pl.BlockSpec((1, D), lambda i: (0, 0)),
      ],
      out_specs=pl.BlockSpec((BM, D), lambda i: (i, 0)),
      out_shape=jax.ShapeDtypeStruct((M, D), jnp.float32),
  )(p, b)


def kernel(feats, edge_index, edge_weight, W1, b1, W2, b2, W3, b3, W4, b4,
           ln_g, ln_b):
  b1r = b1.reshape(1, DH)
  b2r = b2.reshape(1, DH)
  b3r = b3.reshape(1, DH)
  b4r = b4.reshape(1, NCLASS)
  gr = ln_g.reshape(1, DH)
  br = ln_b.reshape(1, DH)

  src_idx = edge_index[0].reshape(NT, NCHUNK, CHUNK)
  dst_idx = edge_index[1].reshape(NT, NCHUNK, CHUNK)
  wbits = jax.lax.bitcast_convert_type(
      edge_weight, jnp.int32).reshape(NT, NCHUNK, CHUNK)
  esd = jnp.stack([src_idx, dst_idx, wbits], axis=2)

  h = _mm_tc(feats, W1)
  p = _segsum_sc(h, esd, DH)
  h = _fuse_tc(p, b1r, gr, br, W2)
  p = _segsum_sc(h, esd, DH)
  h = _fuse_tc(p, b2r, gr, br, W3)
  p = _segsum_sc(h, esd, DH)
  W4p = jnp.pad(W4, ((0, 0), (0, DH - NCLASS)))
  h = _fuse_tc(p, b3r, gr, br, W4p)
  p = _segsum_sc(h, esd, DH)
  return _final_tc(p, b4r)
